# baseline (speedup 1.0000x reference)
import functools

import jax
import jax.numpy as jnp
from jax import lax
from jax.experimental import pallas as pl
from jax.experimental.pallas import tpu as pltpu
from jax.experimental.pallas import tpu_sc as plsc

NC = 2
NS = 16
NW = NC * NS
LANES = 128

_HI = jax.lax.Precision.HIGHEST
_DE = jax.lax.Precision.DEFAULT


def _sc_embed(emb_table, x_idx2d):
    rpt = 8
    n_tiles = x_idx2d.shape[0] // rpt
    assert n_tiles * rpt == x_idx2d.shape[0] and n_tiles <= NW
    h = emb_table.shape[1]
    mesh = plsc.VectorSubcoreMesh(core_axis_name="c", subcore_axis_name="s")

    @functools.partial(
        pl.kernel,
        out_type=jax.ShapeDtypeStruct((x_idx2d.shape[0] * LANES, h),
                                      jnp.float32),
        mesh=mesh,
        scratch_types=[
            pltpu.VMEM((rpt, LANES), jnp.int32),
            pltpu.VMEM((LANES, h), jnp.float32),
            pltpu.SemaphoreType.DMA,
        ],
    )
    def k(emb_hbm, idx_hbm, out_hbm, idx_v, rows_v, sem):
        c = lax.axis_index("c")
        s = lax.axis_index("s")
        t = c * NS + s

        @pl.when(t < n_tiles)
        def _():
            base = t * rpt
            pltpu.sync_copy(idx_hbm.at[pl.ds(base, rpt)], idx_v)
            for j in range(rpt):
                pltpu.async_copy(emb_hbm.at[idx_v.at[j]], rows_v, sem).wait()
                pltpu.sync_copy(rows_v,
                                out_hbm.at[pl.ds((base + j) * LANES, LANES)])

    return k(emb_table, x_idx2d)


def _sc_aggregate(m, src_i2d, dst_i2d, zeros_blk, acc_rows, tr0, tr1):
    h = m.shape[1]
    zr = zeros_blk.shape[0]
    zc = acc_rows // (NS * zr)
    assert acc_rows == NS * zr * zc
    assert tr0 % 16 == 0 and tr1 % 16 == 0
    assert NS * (tr0 + tr1) == src_i2d.shape[0]
    wr = acc_rows // NS
    mesh = plsc.VectorSubcoreMesh(core_axis_name="c", subcore_axis_name="s")

    @functools.partial(
        pl.kernel,
        out_type=jax.ShapeDtypeStruct((NC, acc_rows, h), jnp.float32),
        mesh=mesh,
        scratch_types=[
            pltpu.VMEM((16, LANES), jnp.int32),
            pltpu.VMEM((16, LANES), jnp.int32),
            pltpu.VMEM((LANES, h), jnp.float32),
            pltpu.VMEM((LANES, h), jnp.float32),
            pltpu.VMEM_SHARED((acc_rows, h), jnp.float32),
            pltpu.SemaphoreType.DMA,
            pltpu.SemaphoreType.DMA,
        ],
    )
    def k(m_hbm, src_hbm, dst_hbm, z_hbm, out_hbm,
          src_v, dst_v, rows0, rows1, acc, sem0, sem1):
        c = lax.axis_index("c")
        s = lax.axis_index("s")
        my_tr = jnp.where(c == 0, tr0, tr1)
        base = c * (NS * tr0) + s * my_tr
        slab = 16
        for q in range(zc):
            pltpu.sync_copy(z_hbm, acc.at[pl.ds((s * zc + q) * zr, zr)])
        plsc.subcore_barrier()

        @pl.loop(0, my_tr // slab)
        def _(u):
            sbase = base + u * slab
            pltpu.sync_copy(src_hbm.at[pl.ds(sbase, slab)], src_v)
            pltpu.sync_copy(dst_hbm.at[pl.ds(sbase, slab)], dst_v)
            pltpu.async_copy(m_hbm.at[src_v.at[0]], rows0, sem0)

            @pl.loop(0, slab // 2)
            def _(i):
                j0 = 2 * i
                pltpu.async_copy(m_hbm.at[src_v.at[j0 + 1]], rows1, sem1)
                pltpu.make_async_copy(m_hbm.at[src_v.at[j0]], rows0,
                                      sem0).wait()
                pltpu.sync_copy(rows0, acc.at[dst_v.at[j0]], add=True)

                @pl.when(j0 + 2 < slab)
                def _():
                    pltpu.async_copy(m_hbm.at[src_v.at[j0 + 2]], rows0, sem0)

                pltpu.make_async_copy(m_hbm.at[src_v.at[j0 + 1]], rows1,
                                      sem1).wait()
                pltpu.sync_copy(rows1, acc.at[dst_v.at[j0 + 1]], add=True)

        plsc.subcore_barrier()
        pltpu.sync_copy(acc.at[pl.ds(s * wr, wr)],
                        out_hbm.at[c].at[pl.ds(s * wr, wr)])

    return k(m, src_i2d, dst_i2d, zeros_blk)


def _tc_mg(h, wcat, bvec, n, bn):
    hd = h.shape[1]
    ko = wcat.shape[1]

    def body(h_ref, w_ref, b_ref, m_ref, gh_ref):
        mg = jax.lax.dot_general(
            h_ref[...], w_ref[...], (((1,), (0,)), ((), ())),
            precision=_DE, preferred_element_type=jnp.float32) + b_ref[...]
        m_ref[...] = mg[:, :hd]
        gh_ref[...] = mg[:, hd:]

    return pl.pallas_call(
        body,
        grid=(n // bn,),
        in_specs=[
            pl.BlockSpec((bn, hd), lambda i: (i, 0)),
            pl.BlockSpec((hd, ko), lambda i: (0, 0)),
            pl.BlockSpec((1, ko), lambda i: (0, 0)),
        ],
        out_specs=[
            pl.BlockSpec((bn, hd), lambda i: (i, 0)),
            pl.BlockSpec((bn, ko - hd), lambda i: (i, 0)),
        ],
        out_shape=[
            jax.ShapeDtypeStruct((n, hd), jnp.float32),
            jax.ShapeDtypeStruct((n, ko - hd), jnp.float32),
        ],
    )(h, wcat, bvec)


def _gru_block(aggp, gh, h_prev, w_iht, b_ih):
    agg = aggp[0] + aggp[1]
    hd = h_prev.shape[1]
    gi = jax.lax.dot_general(
        agg, w_iht, (((1,), (0,)), ((), ())),
        precision=_DE, preferred_element_type=jnp.float32) + b_ih
    r = jax.nn.sigmoid(gi[:, :hd] + gh[:, :hd])
    z = jax.nn.sigmoid(gi[:, hd:2 * hd] + gh[:, hd:2 * hd])
    nn = jnp.tanh(gi[:, 2 * hd:] + r * gh[:, 2 * hd:])
    return (1.0 - z) * nn + z * h_prev


def _tc_gru_next(aggp, gh, h, w_iht, b_ih, wcat, bvec, n, bn):
    hd = h.shape[1]
    ko = wcat.shape[1]
    gw = gh.shape[1]

    def body(a_ref, gh_ref, h_ref, wi_ref, bi_ref, wc_ref, bv_ref,
             hn_ref, m_ref, ghn_ref):
        hn = _gru_block(a_ref[...], gh_ref[...], h_ref[...],
                        wi_ref[...], bi_ref[...])
        hn_ref[...] = hn
        mg = jax.lax.dot_general(
            hn, wc_ref[...], (((1,), (0,)), ((), ())),
            precision=_DE, preferred_element_type=jnp.float32) + bv_ref[...]
        m_ref[...] = mg[:, :hd]
        ghn_ref[...] = mg[:, hd:]

    return pl.pallas_call(
        body,
        grid=(n // bn,),
        in_specs=[
            pl.BlockSpec((2, bn, hd), lambda i: (0, i, 0)),
            pl.BlockSpec((bn, gw), lambda i: (i, 0)),
            pl.BlockSpec((bn, hd), lambda i: (i, 0)),
            pl.BlockSpec((hd, gw), lambda i: (0, 0)),
            pl.BlockSpec((1, gw), lambda i: (0, 0)),
            pl.BlockSpec((hd, ko), lambda i: (0, 0)),
            pl.BlockSpec((1, ko), lambda i: (0, 0)),
        ],
        out_specs=[
            pl.BlockSpec((bn, hd), lambda i: (i, 0)),
            pl.BlockSpec((bn, hd), lambda i: (i, 0)),
            pl.BlockSpec((bn, ko - hd), lambda i: (i, 0)),
        ],
        out_shape=[
            jax.ShapeDtypeStruct((n, hd), jnp.float32),
            jax.ShapeDtypeStruct((n, hd), jnp.float32),
            jax.ShapeDtypeStruct((n, ko - hd), jnp.float32),
        ],
    )(aggp, gh, h, w_iht, b_ih, wcat, bvec)


def _tc_gru_pool(aggp, gh, h, w_iht, b_ih, batchf, g, n, bn):
    hd = h.shape[1]
    gw = gh.shape[1]
    nb = n // bn

    def body(a_ref, gh_ref, h_ref, wi_ref, bi_ref, b_ref, out_ref,
             sum_scr, cnt_scr):
        i = pl.program_id(0)

        @pl.when(i == 0)
        def _():
            sum_scr[...] = jnp.zeros_like(sum_scr)
            cnt_scr[...] = jnp.zeros_like(cnt_scr)

        hn = _gru_block(a_ref[...], gh_ref[...], h_ref[...],
                        wi_ref[...], bi_ref[...])
        gids = jax.lax.broadcasted_iota(jnp.int32, (bn, g), 1)
        oh = jnp.where(b_ref[...] == gids, 1.0, 0.0)
        sum_scr[...] += jax.lax.dot_general(
            oh, hn, (((0,), (0,)), ((), ())),
            precision=_HI, preferred_element_type=jnp.float32)
        cnt_scr[...] += jax.lax.dot_general(
            oh, jnp.ones((bn, hd), jnp.float32), (((0,), (0,)), ((), ())),
            precision=_HI, preferred_element_type=jnp.float32)

        @pl.when(i == nb - 1)
        def _():
            out_ref[...] = sum_scr[...] / jnp.maximum(cnt_scr[...], 1.0)

    return pl.pallas_call(
        body,
        grid=(nb,),
        in_specs=[
            pl.BlockSpec((2, bn, hd), lambda i: (0, i, 0)),
            pl.BlockSpec((bn, gw), lambda i: (i, 0)),
            pl.BlockSpec((bn, hd), lambda i: (i, 0)),
            pl.BlockSpec((hd, gw), lambda i: (0, 0)),
            pl.BlockSpec((1, gw), lambda i: (0, 0)),
            pl.BlockSpec((bn, 1), lambda i: (i, 0)),
        ],
        out_specs=pl.BlockSpec((g, hd), lambda i: (0, 0)),
        out_shape=jax.ShapeDtypeStruct((g, hd), jnp.float32),
        scratch_shapes=[
            pltpu.VMEM((g, hd), jnp.float32),
            pltpu.VMEM((g, hd), jnp.float32),
        ],
    )(aggp, gh, h, w_iht, b_ih, batchf)


def kernel(x, edge_index, batch, emb_table, W, w_ih, w_hh, b_ih, b_hh):
    n = x.shape[0]
    hd = emb_table.shape[1]
    num_layers = W.shape[0]
    e = edge_index.shape[1]
    g = 64
    bn = 400

    acc_rows = 10240
    assert acc_rows >= n + 1 and acc_rows % (NS * LANES) == 0
    chunk = 8 * LANES * NW
    ep = ((e + chunk - 1) // chunk) * chunk
    src = edge_index[0]
    dst = edge_index[1]
    pad = ep - e
    src_p = jnp.concatenate([src, jnp.zeros((pad,), jnp.int32)])
    dst_p = jnp.concatenate([dst, jnp.full((pad,), n, jnp.int32)])
    src_i2d = src_p.reshape(ep // LANES, LANES)
    dst_i2d = dst_p.reshape(ep // LANES, LANES)
    zeros_blk = jnp.zeros((acc_rows // (NS * 5), hd), jnp.float32)

    xchunk = 8 * LANES
    xp_len = ((n + xchunk - 1) // xchunk) * xchunk
    x_p = jnp.concatenate([x[:, 0], jnp.zeros((xp_len - n,), jnp.int32)])
    x_i2d = x_p.reshape(xp_len // LANES, LANES)

    w_iht = w_ih.T
    b_ih2 = b_ih.reshape(1, 3 * hd)
    bvec = jnp.concatenate([jnp.zeros((hd,), jnp.float32), b_hh])
    bvec = bvec.reshape(1, hd + 3 * hd)
    wcats = [jnp.concatenate([W[l], w_hh.T], axis=1) for l in range(num_layers)]
    batchf = batch.reshape(n, 1)

    h = _sc_embed(emb_table, x_i2d)
    m, gh = _tc_mg(h, wcats[0], bvec, n, bn)
    for l in range(num_layers):
        aggp = _sc_aggregate(m, src_i2d, dst_i2d, zeros_blk, acc_rows,
                             112, 48)
        if l + 1 < num_layers:
            h, m, gh = _tc_gru_next(aggp, gh, h, w_iht, b_ih2,
                                    wcats[l + 1], bvec, n, bn)
        else:
            out = _tc_gru_pool(aggp, gh, h, w_iht, b_ih2, batchf, g, n, bn)
    return out


# speedup vs baseline: 1.1974x; 1.1974x over previous
import functools

import jax
import jax.numpy as jnp
from jax import lax
from jax.experimental import pallas as pl
from jax.experimental.pallas import tpu as pltpu
from jax.experimental.pallas import tpu_sc as plsc

NC = 2
NS = 16
NW = NC * NS
LANES = 128

_HI = jax.lax.Precision.HIGHEST
_DE = jax.lax.Precision.DEFAULT


def _sc_embed(emb_table, x_idx2d):
    rpt = 8
    n_tiles = x_idx2d.shape[0] // rpt
    assert n_tiles * rpt == x_idx2d.shape[0] and n_tiles <= NW
    h = emb_table.shape[1]
    mesh = plsc.VectorSubcoreMesh(core_axis_name="c", subcore_axis_name="s")

    @functools.partial(
        pl.kernel,
        out_type=jax.ShapeDtypeStruct((x_idx2d.shape[0] * LANES, h),
                                      jnp.float32),
        mesh=mesh,
        scratch_types=[
            pltpu.VMEM((rpt, LANES), jnp.int32),
            pltpu.VMEM((LANES, h), jnp.float32),
            pltpu.SemaphoreType.DMA,
        ],
    )
    def k(emb_hbm, idx_hbm, out_hbm, idx_v, rows_v, sem):
        c = lax.axis_index("c")
        s = lax.axis_index("s")
        t = c * NS + s

        @pl.when(t < n_tiles)
        def _():
            base = t * rpt
            pltpu.sync_copy(idx_hbm.at[pl.ds(base, rpt)], idx_v)
            for j in range(rpt):
                pltpu.async_copy(emb_hbm.at[idx_v.at[j]], rows_v, sem).wait()
                pltpu.sync_copy(rows_v,
                                out_hbm.at[pl.ds((base + j) * LANES, LANES)])

    return k(emb_table, x_idx2d)


def _sc_aggregate(m, src_i2d, dst_i2d, zeros_blk, acc_rows, tr0, tr1):
    h = m.shape[1]
    zr = zeros_blk.shape[0]
    zc = acc_rows // (NS * zr)
    assert acc_rows == NS * zr * zc
    assert tr0 % 16 == 0 and tr1 % 16 == 0
    assert NS * (tr0 + tr1) == src_i2d.shape[0]
    wr = acc_rows // NS
    mesh = plsc.VectorSubcoreMesh(core_axis_name="c", subcore_axis_name="s")

    @functools.partial(
        pl.kernel,
        out_type=jax.ShapeDtypeStruct((NC, acc_rows, h), jnp.float32),
        mesh=mesh,
        scratch_types=[
            pltpu.VMEM((16, LANES), jnp.int32),
            pltpu.VMEM((16, LANES), jnp.int32),
            pltpu.VMEM((LANES, h), jnp.float32),
            pltpu.VMEM((LANES, h), jnp.float32),
            pltpu.VMEM_SHARED((acc_rows, h), jnp.float32),
            pltpu.SemaphoreType.DMA,
            pltpu.SemaphoreType.DMA,
        ],
    )
    def k(m_hbm, src_hbm, dst_hbm, z_hbm, out_hbm,
          src_v, dst_v, rows0, rows1, acc, sem0, sem1):
        c = lax.axis_index("c")
        s = lax.axis_index("s")
        my_tr = jnp.where(c == 0, tr0, tr1)
        base = c * (NS * tr0) + s * my_tr
        slab = 16
        for q in range(zc):
            pltpu.sync_copy(z_hbm, acc.at[pl.ds((s * zc + q) * zr, zr)])
        plsc.subcore_barrier()

        @pl.loop(0, my_tr // slab)
        def _(u):
            sbase = base + u * slab
            pltpu.sync_copy(src_hbm.at[pl.ds(sbase, slab)], src_v)
            pltpu.sync_copy(dst_hbm.at[pl.ds(sbase, slab)], dst_v)
            pltpu.async_copy(m_hbm.at[src_v.at[0]], rows0, sem0)

            @pl.loop(0, slab // 2)
            def _(i):
                j0 = 2 * i
                pltpu.async_copy(m_hbm.at[src_v.at[j0 + 1]], rows1, sem1)
                pltpu.make_async_copy(m_hbm.at[src_v.at[j0]], rows0,
                                      sem0).wait()
                pltpu.sync_copy(rows0, acc.at[dst_v.at[j0]], add=True)

                @pl.when(j0 + 2 < slab)
                def _():
                    pltpu.async_copy(m_hbm.at[src_v.at[j0 + 2]], rows0, sem0)

                pltpu.make_async_copy(m_hbm.at[src_v.at[j0 + 1]], rows1,
                                      sem1).wait()
                pltpu.sync_copy(rows1, acc.at[dst_v.at[j0 + 1]], add=True)

        plsc.subcore_barrier()
        pltpu.sync_copy(acc.at[pl.ds(s * wr, wr)],
                        out_hbm.at[c].at[pl.ds(s * wr, wr)])

    return k(m, src_i2d, dst_i2d, zeros_blk)


def _tc_mg(h, wcat, bvec, n, bn):
    hd = h.shape[1]
    ko = wcat.shape[1]

    def body(h_ref, w_ref, b_ref, m_ref, gh_ref):
        mg = jax.lax.dot_general(
            h_ref[...], w_ref[...], (((1,), (0,)), ((), ())),
            precision=_DE, preferred_element_type=jnp.float32) + b_ref[...]
        m_ref[...] = mg[:, :hd]
        gh_ref[...] = mg[:, hd:]

    return pl.pallas_call(
        body,
        grid=(n // bn,),
        in_specs=[
            pl.BlockSpec((bn, hd), lambda i: (i, 0)),
            pl.BlockSpec((hd, ko), lambda i: (0, 0)),
            pl.BlockSpec((1, ko), lambda i: (0, 0)),
        ],
        out_specs=[
            pl.BlockSpec((bn, hd), lambda i: (i, 0)),
            pl.BlockSpec((bn, ko - hd), lambda i: (i, 0)),
        ],
        out_shape=[
            jax.ShapeDtypeStruct((n, hd), jnp.float32),
            jax.ShapeDtypeStruct((n, ko - hd), jnp.float32),
        ],
    )(h, wcat, bvec)


def _gru_block(aggp, gh, h_prev, w_iht, b_ih):
    agg = aggp[0] + aggp[1]
    hd = h_prev.shape[1]
    gi = jax.lax.dot_general(
        agg, w_iht, (((1,), (0,)), ((), ())),
        precision=_DE, preferred_element_type=jnp.float32) + b_ih
    r = jax.nn.sigmoid(gi[:, :hd] + gh[:, :hd])
    z = jax.nn.sigmoid(gi[:, hd:2 * hd] + gh[:, hd:2 * hd])
    nn = jnp.tanh(gi[:, 2 * hd:] + r * gh[:, 2 * hd:])
    return (1.0 - z) * nn + z * h_prev


def _tc_gru_next(aggp, gh, h, w_iht, b_ih, wcat, bvec, n, bn):
    hd = h.shape[1]
    ko = wcat.shape[1]
    gw = gh.shape[1]

    def body(a_ref, gh_ref, h_ref, wi_ref, bi_ref, wc_ref, bv_ref,
             hn_ref, m_ref, ghn_ref):
        hn = _gru_block(a_ref[...], gh_ref[...], h_ref[...],
                        wi_ref[...], bi_ref[...])
        hn_ref[...] = hn
        mg = jax.lax.dot_general(
            hn, wc_ref[...], (((1,), (0,)), ((), ())),
            precision=_DE, preferred_element_type=jnp.float32) + bv_ref[...]
        m_ref[...] = mg[:, :hd]
        ghn_ref[...] = mg[:, hd:]

    return pl.pallas_call(
        body,
        grid=(n // bn,),
        in_specs=[
            pl.BlockSpec((2, bn, hd), lambda i: (0, i, 0)),
            pl.BlockSpec((bn, gw), lambda i: (i, 0)),
            pl.BlockSpec((bn, hd), lambda i: (i, 0)),
            pl.BlockSpec((hd, gw), lambda i: (0, 0)),
            pl.BlockSpec((1, gw), lambda i: (0, 0)),
            pl.BlockSpec((hd, ko), lambda i: (0, 0)),
            pl.BlockSpec((1, ko), lambda i: (0, 0)),
        ],
        out_specs=[
            pl.BlockSpec((bn, hd), lambda i: (i, 0)),
            pl.BlockSpec((bn, hd), lambda i: (i, 0)),
            pl.BlockSpec((bn, ko - hd), lambda i: (i, 0)),
        ],
        out_shape=[
            jax.ShapeDtypeStruct((n, hd), jnp.float32),
            jax.ShapeDtypeStruct((n, hd), jnp.float32),
            jax.ShapeDtypeStruct((n, ko - hd), jnp.float32),
        ],
    )(aggp, gh, h, w_iht, b_ih, wcat, bvec)


def _tc_gru_pool(aggp, gh, h, w_iht, b_ih, batchf, g, n, bn):
    hd = h.shape[1]
    gw = gh.shape[1]
    nb = n // bn

    def body(a_ref, gh_ref, h_ref, wi_ref, bi_ref, b_ref, out_ref,
             sum_scr, cnt_scr):
        i = pl.program_id(0)

        @pl.when(i == 0)
        def _():
            sum_scr[...] = jnp.zeros_like(sum_scr)
            cnt_scr[...] = jnp.zeros_like(cnt_scr)

        hn = _gru_block(a_ref[...], gh_ref[...], h_ref[...],
                        wi_ref[...], bi_ref[...])
        gids = jax.lax.broadcasted_iota(jnp.int32, (bn, g), 1)
        oh = jnp.where(b_ref[...] == gids, 1.0, 0.0)
        sum_scr[...] += jax.lax.dot_general(
            oh, hn, (((0,), (0,)), ((), ())),
            precision=_HI, preferred_element_type=jnp.float32)
        cnt_scr[...] += jax.lax.dot_general(
            oh, jnp.ones((bn, hd), jnp.float32), (((0,), (0,)), ((), ())),
            precision=_HI, preferred_element_type=jnp.float32)

        @pl.when(i == nb - 1)
        def _():
            out_ref[...] = sum_scr[...] / jnp.maximum(cnt_scr[...], 1.0)

    return pl.pallas_call(
        body,
        grid=(nb,),
        in_specs=[
            pl.BlockSpec((2, bn, hd), lambda i: (0, i, 0)),
            pl.BlockSpec((bn, gw), lambda i: (i, 0)),
            pl.BlockSpec((bn, hd), lambda i: (i, 0)),
            pl.BlockSpec((hd, gw), lambda i: (0, 0)),
            pl.BlockSpec((1, gw), lambda i: (0, 0)),
            pl.BlockSpec((bn, 1), lambda i: (i, 0)),
        ],
        out_specs=pl.BlockSpec((g, hd), lambda i: (0, 0)),
        out_shape=jax.ShapeDtypeStruct((g, hd), jnp.float32),
        scratch_shapes=[
            pltpu.VMEM((g, hd), jnp.float32),
            pltpu.VMEM((g, hd), jnp.float32),
        ],
    )(aggp, gh, h, w_iht, b_ih, batchf)


def kernel(x, edge_index, batch, emb_table, W, w_ih, w_hh, b_ih, b_hh):
    n = x.shape[0]
    hd = emb_table.shape[1]
    num_layers = W.shape[0]
    e = edge_index.shape[1]
    g = 64
    bn = 400

    acc_rows = 10240
    assert acc_rows >= n + 1 and acc_rows % (NS * LANES) == 0
    chunk = 8 * LANES * NW
    ep = ((e + chunk - 1) // chunk) * chunk
    src = edge_index[0]
    dst = edge_index[1]
    pad = ep - e
    src_p = jnp.concatenate([src, jnp.zeros((pad,), jnp.int32)])
    dst_p = jnp.concatenate([dst, jnp.full((pad,), n, jnp.int32)])
    src_i2d = src_p.reshape(ep // LANES, LANES)
    dst_i2d = dst_p.reshape(ep // LANES, LANES)
    zeros_blk = jnp.zeros((acc_rows // (NS * 5), hd), jnp.float32)

    xchunk = 8 * LANES
    xp_len = ((n + xchunk - 1) // xchunk) * xchunk
    x_p = jnp.concatenate([x[:, 0], jnp.zeros((xp_len - n,), jnp.int32)])
    x_i2d = x_p.reshape(xp_len // LANES, LANES)

    w_iht = w_ih.T
    b_ih2 = b_ih.reshape(1, 3 * hd)
    bvec = jnp.concatenate([jnp.zeros((hd,), jnp.float32), b_hh])
    bvec = bvec.reshape(1, hd + 3 * hd)
    wcats = [jnp.concatenate([W[l], w_hh.T], axis=1) for l in range(num_layers)]
    batchf = batch.reshape(n, 1)

    h = _sc_embed(emb_table, x_i2d)
    m, gh = _tc_mg(h, wcats[0], bvec, n, bn)
    for l in range(num_layers):
        aggp = _sc_aggregate(m, src_i2d, dst_i2d, zeros_blk, acc_rows,
                             144, 16)
        if l + 1 < num_layers:
            h, m, gh = _tc_gru_next(aggp, gh, h, w_iht, b_ih2,
                                    wcats[l + 1], bvec, n, bn)
        else:
            out = _tc_gru_pool(aggp, gh, h, w_iht, b_ih2, batchf, g, n, bn)
    return out
